# baseline (speedup 1.0000x reference)
import jax
import jax.numpy as jnp
from jax.experimental import pallas as pl

B, N, D = 16, 4096, 64
K = 4

def _body(feat_ref, out_ref):
    out_ref[...] = feat_ref[...]

def kernel(feat, num_unit, v, g, b):
    ft = jnp.transpose(feat, (0, 2, 1))
    out = pl.pallas_call(
        _body,
        grid=(B, K),
        in_specs=[pl.BlockSpec((1, D, N // K), lambda i, k: (i, 0, k))],
        out_specs=pl.BlockSpec((1, D, N // K), lambda i, k: (i, 0, k)),
        out_shape=jax.ShapeDtypeStruct((B, D, N), jnp.float32),
    )(ft)
    return jnp.transpose(out, (0, 2, 1))


# speedup vs baseline: 2.5925x; 2.5925x over previous
import jax
import jax.numpy as jnp
from jax import lax
from jax.experimental import pallas as pl
from jax.experimental.pallas import tpu as pltpu

B, N, D_IN, D_OUT = 16, 4096, 64, 64
SEG = 4
SROWS = D_IN // SEG
NSLOT = 4


def _body(nu_ref, g_ref, feat_hbm, v_ref, b_ref, out_hbm,
          inbuf, outbuf, fsem, osem):
    def in_copy(b, slot, s):
        return pltpu.make_async_copy(
            feat_hbm.at[b, pl.ds(s * SROWS, SROWS), :],
            inbuf.at[slot, pl.ds(s * SROWS, SROWS), :],
            fsem.at[slot, s])

    def out_copy(b, slot, s):
        return pltpu.make_async_copy(
            outbuf.at[slot, pl.ds(s * SROWS, SROWS), :],
            out_hbm.at[b, pl.ds(s * SROWS, SROWS), :],
            osem.at[slot, s])

    v = v_ref[...]
    norm = jnp.sqrt(jnp.sum(v * v))
    w = v * (g_ref[0] / norm)
    bias = b_ref[...]
    lane = lax.broadcasted_iota(jnp.int32, (1, N), 1)

    for b in range(min(NSLOT - 1, B)):
        for s in range(SEG):
            in_copy(b, b % NSLOT, s).start()

    for b in range(B):
        slot = b % NSLOT
        nxt = b + NSLOT - 1
        if nxt < B:
            for s in range(SEG):
                in_copy(nxt, nxt % NSLOT, s).start()
        for s in range(SEG):
            in_copy(b, slot, s).wait()
        if b >= NSLOT:
            for s in range(SEG):
                out_copy(b - NSLOT, slot, s).wait()
        x = inbuf[slot]
        pen = jnp.where(lane < nu_ref[b], 0.0, -jnp.inf)
        fmax = jnp.max(x + pen, axis=1, keepdims=True)
        adj = bias - lax.dot_general(w, fmax, (((1,), (0,)), ((), ())),
                                     preferred_element_type=jnp.float32)
        out = lax.dot_general(w, x, (((1,), (0,)), ((), ())),
                              preferred_element_type=jnp.float32)
        outbuf[slot] = jnp.maximum(out + adj, 0.0)
        for s in range(SEG):
            out_copy(b, slot, s).start()

    for b in range(max(B - NSLOT, 0), B):
        for s in range(SEG):
            out_copy(b, b % NSLOT, s).wait()


def kernel(feat, num_unit, v, g, b):
    ft = jnp.transpose(feat, (0, 2, 1))
    g2 = jnp.reshape(g, (1,))
    b2 = jnp.reshape(b, (D_OUT, 1))
    out_t = pl.pallas_call(
        _body,
        grid=(),
        in_specs=[
            pl.BlockSpec(memory_space=pltpu.SMEM),
            pl.BlockSpec(memory_space=pltpu.SMEM),
            pl.BlockSpec(memory_space=pl.ANY),
            pl.BlockSpec(memory_space=pltpu.VMEM),
            pl.BlockSpec(memory_space=pltpu.VMEM),
        ],
        out_specs=pl.BlockSpec(memory_space=pl.ANY),
        out_shape=jax.ShapeDtypeStruct((B, D_OUT, N), jnp.float32),
        scratch_shapes=[
            pltpu.VMEM((NSLOT, D_IN, N), jnp.float32),
            pltpu.VMEM((NSLOT, D_OUT, N), jnp.float32),
            pltpu.SemaphoreType.DMA((NSLOT, SEG)),
            pltpu.SemaphoreType.DMA((NSLOT, SEG)),
        ],
    )(num_unit, g2, ft, v, b2)
    return jnp.transpose(out_t, (0, 2, 1))


# speedup vs baseline: 2.6841x; 1.0353x over previous
import jax
import jax.numpy as jnp
from jax import lax
from jax.experimental import pallas as pl
from jax.experimental.pallas import tpu as pltpu

B, N, D_IN, D_OUT = 16, 4096, 64, 64
SEG = 8
SROWS = D_IN // SEG
NSLOT = 6


def _body(nu_ref, g_ref, feat_hbm, v_ref, b_ref, out_hbm,
          inbuf, outbuf, fsem, osem):
    def in_copy(b, slot, s):
        return pltpu.make_async_copy(
            feat_hbm.at[b, pl.ds(s * SROWS, SROWS), :],
            inbuf.at[slot, pl.ds(s * SROWS, SROWS), :],
            fsem.at[slot, s])

    def out_copy(b, slot, s):
        return pltpu.make_async_copy(
            outbuf.at[slot, pl.ds(s * SROWS, SROWS), :],
            out_hbm.at[b, pl.ds(s * SROWS, SROWS), :],
            osem.at[slot, s])

    v = v_ref[...]
    norm = jnp.sqrt(jnp.sum(v * v))
    w = v * (g_ref[0] / norm)
    bias = b_ref[...]
    lane = lax.broadcasted_iota(jnp.int32, (1, N), 1)

    for b in range(min(NSLOT - 1, B)):
        for s in range(SEG):
            in_copy(b, b % NSLOT, s).start()

    for b in range(B):
        slot = b % NSLOT
        nxt = b + NSLOT - 1
        if nxt < B:
            for s in range(SEG):
                in_copy(nxt, nxt % NSLOT, s).start()
        for s in range(SEG):
            in_copy(b, slot, s).wait()
        if b >= NSLOT:
            for s in range(SEG):
                out_copy(b - NSLOT, slot, s).wait()
        x = inbuf[slot]
        pen = jnp.where(lane < nu_ref[b], 0.0, -jnp.inf)
        fmax = jnp.max(x + pen, axis=1, keepdims=True)
        adj = bias - lax.dot_general(w, fmax, (((1,), (0,)), ((), ())),
                                     preferred_element_type=jnp.float32)
        out = lax.dot_general(w, x, (((1,), (0,)), ((), ())),
                              preferred_element_type=jnp.float32)
        outbuf[slot] = jnp.maximum(out + adj, 0.0)
        for s in range(SEG):
            out_copy(b, slot, s).start()

    for b in range(max(B - NSLOT, 0), B):
        for s in range(SEG):
            out_copy(b, b % NSLOT, s).wait()


def kernel(feat, num_unit, v, g, b):
    ft = jnp.transpose(feat, (0, 2, 1))
    g2 = jnp.reshape(g, (1,))
    b2 = jnp.reshape(b, (D_OUT, 1))
    out_t = pl.pallas_call(
        _body,
        grid=(),
        in_specs=[
            pl.BlockSpec(memory_space=pltpu.SMEM),
            pl.BlockSpec(memory_space=pltpu.SMEM),
            pl.BlockSpec(memory_space=pl.ANY),
            pl.BlockSpec(memory_space=pltpu.VMEM),
            pl.BlockSpec(memory_space=pltpu.VMEM),
        ],
        out_specs=pl.BlockSpec(memory_space=pl.ANY),
        out_shape=jax.ShapeDtypeStruct((B, D_OUT, N), jnp.float32),
        scratch_shapes=[
            pltpu.VMEM((NSLOT, D_IN, N), jnp.float32),
            pltpu.VMEM((NSLOT, D_OUT, N), jnp.float32),
            pltpu.SemaphoreType.DMA((NSLOT, SEG)),
            pltpu.SemaphoreType.DMA((NSLOT, SEG)),
        ],
    )(num_unit, g2, ft, v, b2)
    return jnp.transpose(out_t, (0, 2, 1))


# speedup vs baseline: 2.7695x; 1.0318x over previous
import jax
import jax.numpy as jnp
from jax import lax
from jax.experimental import pallas as pl
from jax.experimental.pallas import tpu as pltpu

B, N, D_IN, D_OUT = 16, 4096, 64, 64
SEG = 4
SROWS = D_IN // SEG
NSLOT = 16


def _body(nu_ref, g_ref, feat_hbm, v_ref, b_ref, out_hbm,
          inbuf, outbuf, fsem, osem):
    def in_copy(b, slot, s):
        return pltpu.make_async_copy(
            feat_hbm.at[b, pl.ds(s * SROWS, SROWS), :],
            inbuf.at[slot, pl.ds(s * SROWS, SROWS), :],
            fsem.at[slot, s])

    def out_copy(b, slot, s):
        return pltpu.make_async_copy(
            outbuf.at[slot, pl.ds(s * SROWS, SROWS), :],
            out_hbm.at[b, pl.ds(s * SROWS, SROWS), :],
            osem.at[slot, s])

    v = v_ref[...]
    norm = jnp.sqrt(jnp.sum(v * v))
    w = v * (g_ref[0] / norm)
    bias = b_ref[...]
    lane = lax.broadcasted_iota(jnp.int32, (1, N), 1)

    for b in range(min(NSLOT - 1, B)):
        for s in range(SEG):
            in_copy(b, b % NSLOT, s).start()

    for b in range(B):
        slot = b % NSLOT
        nxt = b + NSLOT - 1
        if nxt < B:
            for s in range(SEG):
                in_copy(nxt, nxt % NSLOT, s).start()
        for s in range(SEG):
            in_copy(b, slot, s).wait()
        if b >= NSLOT:
            for s in range(SEG):
                out_copy(b - NSLOT, slot, s).wait()
        x = inbuf[slot]
        pen = jnp.where(lane < nu_ref[b], 0.0, -jnp.inf)
        fmax = jnp.max(x + pen, axis=1, keepdims=True)
        adj = bias - lax.dot_general(w, fmax, (((1,), (0,)), ((), ())),
                                     preferred_element_type=jnp.float32)
        out = lax.dot_general(w, x, (((1,), (0,)), ((), ())),
                              preferred_element_type=jnp.float32)
        outbuf[slot] = jnp.maximum(out + adj, 0.0)
        for s in range(SEG):
            out_copy(b, slot, s).start()

    for b in range(max(B - NSLOT, 0), B):
        for s in range(SEG):
            out_copy(b, b % NSLOT, s).wait()


def kernel(feat, num_unit, v, g, b):
    ft = jnp.transpose(feat, (0, 2, 1))
    g2 = jnp.reshape(g, (1,))
    b2 = jnp.reshape(b, (D_OUT, 1))
    out_t = pl.pallas_call(
        _body,
        grid=(),
        in_specs=[
            pl.BlockSpec(memory_space=pltpu.SMEM),
            pl.BlockSpec(memory_space=pltpu.SMEM),
            pl.BlockSpec(memory_space=pl.ANY),
            pl.BlockSpec(memory_space=pltpu.VMEM),
            pl.BlockSpec(memory_space=pltpu.VMEM),
        ],
        out_specs=pl.BlockSpec(memory_space=pl.ANY),
        out_shape=jax.ShapeDtypeStruct((B, D_OUT, N), jnp.float32),
        scratch_shapes=[
            pltpu.VMEM((NSLOT, D_IN, N), jnp.float32),
            pltpu.VMEM((NSLOT, D_OUT, N), jnp.float32),
            pltpu.SemaphoreType.DMA((NSLOT, SEG)),
            pltpu.SemaphoreType.DMA((NSLOT, SEG)),
        ],
    )(num_unit, g2, ft, v, b2)
    return jnp.transpose(out_t, (0, 2, 1))
